# baseline (speedup 1.0000x reference)
import functools

import jax
import jax.numpy as jnp
from jax import lax
from jax.experimental import pallas as pl
from jax.experimental.pallas import tpu as pltpu
from jax.experimental.pallas import tpu_sc as plsc

_CHUNK = 128


def _sc_geometry():
    info = plsc.get_sparse_core_info()
    return info.num_cores, info.num_subcores


@functools.partial(jax.jit, static_argnames=("num_cores", "num_subcores"))
def _lookup(indices, table, num_cores, num_subcores):
    num_workers = num_cores * num_subcores
    batch, width = indices.shape[0], table.shape[1]
    b_per_w = batch // num_workers
    n_chunks = b_per_w // _CHUNK

    mesh = plsc.VectorSubcoreMesh(core_axis_name="c", subcore_axis_name="s")

    @functools.partial(
        pl.kernel,
        out_type=jax.ShapeDtypeStruct((batch, width), table.dtype),
        mesh=mesh,
        scratch_types=[
            pltpu.VMEM((n_chunks, _CHUNK), jnp.int32),
            pltpu.VMEM((b_per_w, width), table.dtype),
            pltpu.SemaphoreType.DMA,
        ],
        compiler_params=pltpu.CompilerParams(use_tc_tiling_on_sc=False),
    )
    def gather_kernel(idx_hbm, table_hbm, out_hbm, idx_v, rows_v, sem):
        wid = lax.axis_index("s") * num_cores + lax.axis_index("c")
        base = wid * b_per_w
        for j in range(n_chunks):
            pltpu.sync_copy(
                idx_hbm.at[pl.ds(base + j * _CHUNK, _CHUNK)], idx_v.at[j]
            )
        copies = [
            pltpu.async_copy(
                table_hbm.at[idx_v.at[j]],
                rows_v.at[pl.ds(j * _CHUNK, _CHUNK)],
                sem,
            )
            for j in range(n_chunks)
        ]
        for c in copies:
            c.wait()
        pltpu.sync_copy(rows_v, out_hbm.at[pl.ds(base, b_per_w)])

    return gather_kernel(indices, table)


def kernel(indices, W_relation):
    num_cores, num_subcores = _sc_geometry()
    return _lookup(
        indices.astype(jnp.int32), W_relation, num_cores, num_subcores
    )


# speedup vs baseline: 1.5136x; 1.5136x over previous
import functools

import jax
import jax.numpy as jnp
from jax import lax
from jax.experimental import pallas as pl
from jax.experimental.pallas import tpu as pltpu
from jax.experimental.pallas import tpu_sc as plsc


def _sc_geometry():
    info = plsc.get_sparse_core_info()
    return info.num_cores, info.num_subcores


@functools.partial(jax.jit, static_argnames=("num_cores", "num_subcores"))
def _lookup(indices, table, num_cores, num_subcores):
    num_workers = num_cores * num_subcores
    batch, width = indices.shape[0], table.shape[1]
    b_per_w = batch // num_workers

    mesh = plsc.VectorSubcoreMesh(core_axis_name="c", subcore_axis_name="s")

    @functools.partial(
        pl.kernel,
        out_type=jax.ShapeDtypeStruct((batch, width), table.dtype),
        mesh=mesh,
        scratch_types=[
            pltpu.VMEM((b_per_w,), jnp.int32),
            pltpu.VMEM((b_per_w, width), table.dtype),
            pltpu.SemaphoreType.DMA,
        ],
    )
    def gather_kernel(idx_hbm, table_hbm, out_hbm, idx_v, rows_v, sem):
        wid = lax.axis_index("s") * num_cores + lax.axis_index("c")
        base = wid * b_per_w
        pltpu.sync_copy(idx_hbm.at[pl.ds(base, b_per_w)], idx_v)

        def body(i, carry):
            vec = idx_v[pl.ds(i * 16, 16)]
            for k in range(16):
                row = vec[k]
                pltpu.async_copy(
                    table_hbm.at[pl.ds(row, 1), :],
                    rows_v.at[pl.ds(i * 16 + k, 1), :],
                    sem,
                )
            return carry

        lax.fori_loop(0, b_per_w // 16, body, 0)
        pltpu.make_async_copy(
            table_hbm.at[pl.ds(0, b_per_w), :], rows_v, sem
        ).wait()
        pltpu.sync_copy(rows_v, out_hbm.at[pl.ds(base, b_per_w)])

    return gather_kernel(indices, table)


def kernel(indices, W_relation):
    num_cores, num_subcores = _sc_geometry()
    return _lookup(
        indices.astype(jnp.int32), W_relation, num_cores, num_subcores
    )


# speedup vs baseline: 1.9403x; 1.2819x over previous
import functools

import jax
import jax.numpy as jnp
from jax import lax
from jax.experimental import pallas as pl
from jax.experimental.pallas import tpu as pltpu
from jax.experimental.pallas import tpu_sc as plsc

_LANES = 16
_OUT_CHUNK = 4096


def _sc_geometry():
    info = plsc.get_sparse_core_info()
    return info.num_cores, info.num_subcores


@functools.partial(jax.jit, static_argnames=("num_cores", "num_subcores"))
def _lookup(indices, table, num_cores, num_subcores):
    num_workers = num_cores * num_subcores
    batch = indices.shape[0]
    table_t = table.T
    width, relations = table_t.shape
    rows_per_w = width // num_workers

    mesh = plsc.VectorSubcoreMesh(core_axis_name="c", subcore_axis_name="s")

    @functools.partial(
        pl.kernel,
        out_type=jax.ShapeDtypeStruct((width, batch), table.dtype),
        mesh=mesh,
        scratch_types=[
            pltpu.VMEM((relations,), table.dtype),
            pltpu.VMEM((batch,), jnp.int32),
            pltpu.VMEM((_OUT_CHUNK,), table.dtype),
        ],
        compiler_params=pltpu.CompilerParams(needs_layout_passes=False),
    )
    def gather_kernel(idx_hbm, table_hbm, out_hbm, row_v, idx_v, out_v):
        wid = lax.axis_index("s") * num_cores + lax.axis_index("c")
        pltpu.sync_copy(idx_hbm, idx_v)
        for p in range(rows_per_w):
            d = wid * rows_per_w + p
            pltpu.sync_copy(table_hbm.at[d], row_v)
            for cb in range(batch // _OUT_CHUNK):

                def body(k, carry, _cb=cb):
                    iv = idx_v[pl.ds(_cb * _OUT_CHUNK + k * _LANES, _LANES)]
                    vals = plsc.load_gather(row_v, [iv])
                    out_v[pl.ds(k * _LANES, _LANES)] = vals
                    return carry

                lax.fori_loop(0, _OUT_CHUNK // _LANES, body, 0)
                pltpu.sync_copy(
                    out_v,
                    out_hbm.at[d, pl.ds(cb * _OUT_CHUNK, _OUT_CHUNK)],
                )

    out_t = gather_kernel(indices, table_t)
    return out_t.T


def kernel(indices, W_relation):
    num_cores, num_subcores = _sc_geometry()
    return _lookup(
        indices.astype(jnp.int32), W_relation, num_cores, num_subcores
    )


# speedup vs baseline: 2.2093x; 1.1387x over previous
import functools

import jax
import jax.numpy as jnp
from jax import lax
from jax.experimental import pallas as pl
from jax.experimental.pallas import tpu as pltpu
from jax.experimental.pallas import tpu_sc as plsc

_LANES = 16
_IDX_CHUNK = 8192
_UNROLL = 8


def _sc_geometry():
    info = plsc.get_sparse_core_info()
    return info.num_cores, info.num_subcores


@functools.partial(jax.jit, static_argnames=("num_cores", "num_subcores"))
def _lookup(indices, table, num_cores, num_subcores):
    num_workers = num_cores * num_subcores
    batch = indices.shape[0]
    table_t = table.T
    width, relations = table_t.shape
    rows_per_w = width // num_workers

    mesh = plsc.VectorSubcoreMesh(core_axis_name="c", subcore_axis_name="s")

    @functools.partial(
        pl.kernel,
        out_type=jax.ShapeDtypeStruct((width, batch), table.dtype),
        mesh=mesh,
        scratch_types=[
            pltpu.VMEM((relations,), table.dtype),
            pltpu.VMEM((_IDX_CHUNK,), jnp.int32),
            pltpu.VMEM((batch,), table.dtype),
        ],
        compiler_params=pltpu.CompilerParams(needs_layout_passes=False),
    )
    def gather_kernel(idx_hbm, table_hbm, out_hbm, row_v, idx_v, out_v):
        wid = lax.axis_index("s") * num_cores + lax.axis_index("c")
        group = _UNROLL * _LANES
        for p in range(rows_per_w):
            d = wid * rows_per_w + p
            pltpu.sync_copy(table_hbm.at[d], row_v)
            for h in range(batch // _IDX_CHUNK):
                pltpu.sync_copy(
                    idx_hbm.at[pl.ds(h * _IDX_CHUNK, _IDX_CHUNK)], idx_v
                )

                def body(k, carry, _h=h):
                    for u in range(_UNROLL):
                        off = k * group + u * _LANES
                        iv = idx_v[pl.ds(off, _LANES)]
                        vals = plsc.load_gather(row_v, [iv])
                        out_v[pl.ds(_h * _IDX_CHUNK + off, _LANES)] = vals
                    return carry

                lax.fori_loop(0, _IDX_CHUNK // group, body, 0)
            pltpu.sync_copy(out_v, out_hbm.at[d])

    out_t = gather_kernel(indices, table_t)
    return out_t.T


def kernel(indices, W_relation):
    num_cores, num_subcores = _sc_geometry()
    return _lookup(
        indices.astype(jnp.int32), W_relation, num_cores, num_subcores
    )


# speedup vs baseline: 2.3815x; 1.0779x over previous
import functools

import jax
import jax.numpy as jnp
from jax import lax
from jax.experimental import pallas as pl
from jax.experimental.pallas import tpu as pltpu
from jax.experimental.pallas import tpu_sc as plsc

_LANES = 16
_OUT_CHUNK = 4096
_UNROLL = 8


def _sc_geometry():
    info = plsc.get_sparse_core_info()
    return info.num_cores, info.num_subcores


@functools.partial(jax.jit, static_argnames=("num_cores", "num_subcores"))
def _lookup(indices, table, num_cores, num_subcores):
    num_workers = num_cores * num_subcores
    batch = indices.shape[0]
    table_t = table.T
    width, relations = table_t.shape
    rows_per_w = width // num_workers

    mesh = plsc.VectorSubcoreMesh(core_axis_name="c", subcore_axis_name="s")

    @functools.partial(
        pl.kernel,
        out_type=jax.ShapeDtypeStruct((width, batch), table.dtype),
        mesh=mesh,
        scratch_types=[
            pltpu.VMEM((relations,), table.dtype),
            pltpu.VMEM((batch,), jnp.int32),
            pltpu.VMEM((_OUT_CHUNK,), table.dtype),
            pltpu.VMEM((_OUT_CHUNK,), table.dtype),
            pltpu.SemaphoreType.DMA,
            pltpu.SemaphoreType.DMA,
            pltpu.SemaphoreType.DMA,
            pltpu.SemaphoreType.DMA,
        ],
        compiler_params=pltpu.CompilerParams(needs_layout_passes=False),
    )
    def gather_kernel(
        idx_hbm, table_hbm, out_hbm,
        row_v, idx_v, out_v0, out_v1,
        sem_row, sem_idx, sem_o0, sem_o1,
    ):
        wid = lax.axis_index("s") * num_cores + lax.axis_index("c")
        group = _UNROLL * _LANES
        n_chunks = batch // _OUT_CHUNK
        outs = [out_v0, out_v1]
        sems = [sem_o0, sem_o1]
        pending = [None, None]
        row_copy = None

        for p in range(rows_per_w):
            d = wid * rows_per_w + p
            if p == 0:
                idx_copy = pltpu.async_copy(idx_hbm, idx_v, sem_idx)
                row_copy = pltpu.async_copy(table_hbm.at[d], row_v, sem_row)
                idx_copy.wait()
            row_copy.wait()
            for h in range(n_chunks):
                b = h % 2
                if pending[b] is not None:
                    pending[b].wait()

                def body(k, carry, _h=h, _b=b):
                    for u in range(_UNROLL):
                        off = k * group + u * _LANES
                        iv = idx_v[pl.ds(_h * _OUT_CHUNK + off, _LANES)]
                        vals = plsc.load_gather(row_v, [iv])
                        outs[_b][pl.ds(off, _LANES)] = vals
                    return carry

                lax.fori_loop(0, _OUT_CHUNK // group, body, 0)
                if p + 1 == rows_per_w and h + 2 >= n_chunks:
                    pltpu.sync_copy(
                        outs[b],
                        out_hbm.at[d, pl.ds(h * _OUT_CHUNK, _OUT_CHUNK)],
                    )
                    pending[b] = None
                else:
                    pending[b] = pltpu.async_copy(
                        outs[b],
                        out_hbm.at[d, pl.ds(h * _OUT_CHUNK, _OUT_CHUNK)],
                        sems[b],
                    )
            if p + 1 < rows_per_w:
                row_copy = pltpu.async_copy(
                    table_hbm.at[d + 1], row_v, sem_row
                )

    out_t = gather_kernel(indices, table_t)
    return out_t.T


def kernel(indices, W_relation):
    num_cores, num_subcores = _sc_geometry()
    return _lookup(
        indices.astype(jnp.int32), W_relation, num_cores, num_subcores
    )


# speedup vs baseline: 2.7861x; 1.1699x over previous
import functools

import jax
import jax.numpy as jnp
from jax import lax
from jax.experimental import pallas as pl
from jax.experimental.pallas import tpu as pltpu
from jax.experimental.pallas import tpu_sc as plsc

_LANES = 16
_OUT_CHUNK = 4096
_UNROLL = 8


def _sc_geometry():
    info = plsc.get_sparse_core_info()
    return info.num_cores, info.num_subcores


@functools.partial(jax.jit, static_argnames=("num_cores", "num_subcores"))
def _lookup(indices, table, num_cores, num_subcores):
    num_workers = num_cores * num_subcores
    batch = indices.shape[0]
    table_t = table.T
    width, relations = table_t.shape
    rows_per_w = width // num_workers

    mesh = plsc.VectorSubcoreMesh(core_axis_name="c", subcore_axis_name="s")

    @functools.partial(
        pl.kernel,
        out_type=jax.ShapeDtypeStruct((width, batch), table.dtype),
        mesh=mesh,
        scratch_types=[
            pltpu.VMEM((relations,), table.dtype),
            pltpu.VMEM((batch,), jnp.int32),
            pltpu.VMEM((_OUT_CHUNK,), table.dtype),
            pltpu.VMEM((_OUT_CHUNK,), table.dtype),
            pltpu.SemaphoreType.DMA,
            pltpu.SemaphoreType.DMA,
            pltpu.SemaphoreType.DMA,
            pltpu.SemaphoreType.DMA,
        ],
        compiler_params=pltpu.CompilerParams(needs_layout_passes=False),
    )
    def gather_kernel(
        idx_hbm, table_hbm, out_hbm,
        row_v, idx_v, out_v0, out_v1,
        sem_row, sem_idx, sem_o0, sem_o1,
    ):
        wid = lax.axis_index("s") * num_cores + lax.axis_index("c")
        group = _UNROLL * _LANES
        n_chunks = batch // _OUT_CHUNK
        outs = [out_v0, out_v1]
        sems = [sem_o0, sem_o1]
        pending = [None, None]
        row_copy = None

        for p in range(rows_per_w):
            d = wid * rows_per_w + p
            if p == 0:
                idx_copy = pltpu.async_copy(idx_hbm, idx_v, sem_idx)
                row_copy = pltpu.async_copy(table_hbm.at[d], row_v, sem_row)
                idx_copy.wait()
            row_copy.wait()
            for h in range(n_chunks):
                b = h % 2
                if pending[b] is not None:
                    pending[b].wait()

                @plsc.parallel_loop(0, _OUT_CHUNK, step=_LANES, unroll=_UNROLL)
                def body(off, _h=h, _b=b):
                    iv = idx_v[pl.ds(_h * _OUT_CHUNK + off, _LANES)]
                    outs[_b][pl.ds(off, _LANES)] = plsc.load_gather(
                        row_v, [iv]
                    )
                if p + 1 == rows_per_w and h + 2 >= n_chunks:
                    pltpu.sync_copy(
                        outs[b],
                        out_hbm.at[d, pl.ds(h * _OUT_CHUNK, _OUT_CHUNK)],
                    )
                    pending[b] = None
                else:
                    pending[b] = pltpu.async_copy(
                        outs[b],
                        out_hbm.at[d, pl.ds(h * _OUT_CHUNK, _OUT_CHUNK)],
                        sems[b],
                    )
            if p + 1 < rows_per_w:
                row_copy = pltpu.async_copy(
                    table_hbm.at[d + 1], row_v, sem_row
                )

    out_t = gather_kernel(indices, table_t)
    return out_t.T


def kernel(indices, W_relation):
    num_cores, num_subcores = _sc_geometry()
    return _lookup(
        indices.astype(jnp.int32), W_relation, num_cores, num_subcores
    )
